# baseline (speedup 1.0000x reference)
import functools

import jax
import jax.numpy as jnp
from jax import lax
from jax.experimental import pallas as pl
from jax.experimental.pallas import tpu as pltpu
from jax.experimental.pallas import tpu_sc as plsc

B = 64
N = 1024
IN_FEAT = 192
HIDDEN = 128
CODE_DIM = 32
CODEBOOK_SIZE = 256
OUT_FEAT = 192

TB = 2048
BN = B * N
GRID = BN // TB

NC = 2
NS = 16
NW = NC * NS
TOK_PER_W = BN // NW
CHUNK = 256
NCHUNK = TOK_PER_W // CHUNK


def _tc_body(x_ref, wd_ref, bd_ref, wi_ref, bi_ref, cb_ref, wo_ref, bo_ref,
             wu_ref, bu_ref, idx_ref, table_ref):
    def mm(a, b, dims):
        return jax.lax.dot_general(a.astype(jnp.bfloat16),
                                   b.astype(jnp.bfloat16), dims,
                                   preferred_element_type=jnp.float32)

    @pl.when(pl.program_id(0) == 0)
    def _():
        cb0 = cb_ref[...]
        t0 = mm(cb0, wo_ref[...], (((1,), (1,)), ((), ()))) + bo_ref[...]
        t1 = mm(t0, wu_ref[...], (((1,), (1,)), ((), ()))) + bu_ref[...]
        table_ref[...] = jnp.clip(t1, -1.0, 1.0)

    x = x_ref[...]
    h = mm(x, wd_ref[...], (((1,), (1,)), ((), ()))) + bd_ref[...]
    z = mm(h, wi_ref[...], (((1,), (1,)), ((), ()))) + bi_ref[...]
    cb = cb_ref[...]
    scores = mm(z, cb, (((1,), (1,)), ((), ())))
    zz = jnp.sum(z * z, axis=1, keepdims=True)
    c2 = jax.lax.dot_general(jnp.ones((1, CODE_DIM), jnp.float32), cb * cb,
                             (((1,), (1,)), ((), ())),
                             precision=jax.lax.Precision.HIGHEST,
                             preferred_element_type=jnp.float32)
    dist = (zz - 2.0 * scores) + c2
    dmin = jnp.min(dist, axis=1, keepdims=True)
    lane = jax.lax.broadcasted_iota(jnp.int32, dist.shape, 1)
    idx_ref[...] = jnp.min(jnp.where(dist == dmin, lane, CODEBOOK_SIZE),
                           axis=1, keepdims=True)


def _indices_and_table(xf, W_down, b_down, W_in, b_in, codebook, W_out, b_out,
                       W_up, b_up):
    full = lambda shape: pl.BlockSpec(shape, lambda i: (0,) * len(shape))
    return pl.pallas_call(
        _tc_body,
        grid=(GRID,),
        in_specs=[
            pl.BlockSpec((TB, IN_FEAT), lambda i: (i, 0)),
            full((HIDDEN, IN_FEAT)),
            full((1, HIDDEN)),
            full((CODE_DIM, HIDDEN)),
            full((1, CODE_DIM)),
            full((CODEBOOK_SIZE, CODE_DIM)),
            full((HIDDEN, CODE_DIM)),
            full((1, HIDDEN)),
            full((OUT_FEAT, HIDDEN)),
            full((1, OUT_FEAT)),
        ],
        out_specs=[
            pl.BlockSpec((TB, 1), lambda i: (i, 0)),
            full((CODEBOOK_SIZE, OUT_FEAT)),
        ],
        out_shape=[
            jax.ShapeDtypeStruct((BN, 1), jnp.int32),
            jax.ShapeDtypeStruct((CODEBOOK_SIZE, OUT_FEAT), jnp.float32),
        ],
    )(xf, W_down, b_down.reshape(1, HIDDEN), W_in, b_in.reshape(1, CODE_DIM),
      codebook, W_out, b_out.reshape(1, HIDDEN), W_up,
      b_up.reshape(1, OUT_FEAT))


@functools.partial(
    pl.kernel,
    out_type=jax.ShapeDtypeStruct((BN, OUT_FEAT), jnp.float32),
    mesh=plsc.VectorSubcoreMesh(core_axis_name="c", subcore_axis_name="s",
                                num_cores=NC, num_subcores=NS),
    compiler_params=pltpu.CompilerParams(use_tc_tiling_on_sc=False),
    scratch_types=[
        pltpu.VMEM((CHUNK,), jnp.int32),
        pltpu.VMEM((CHUNK,), jnp.int32),
        pltpu.VMEM((CHUNK, OUT_FEAT), jnp.float32),
        pltpu.VMEM((CHUNK, OUT_FEAT), jnp.float32),
        pltpu.SemaphoreType.DMA,
        pltpu.SemaphoreType.DMA,
    ],
)
def _sc_gather(table_hbm, idx_hbm, out_hbm, idx_a, idx_b, rows_a, rows_b,
               sem_a, sem_b):
    wid = lax.axis_index("s") * NC + lax.axis_index("c")
    base = wid * TOK_PER_W
    bufs = ((idx_a, rows_a, sem_a), (idx_b, rows_b, sem_b))

    pltpu.sync_copy(idx_hbm.at[pl.ds(base, CHUNK)], idx_a)
    copies = [pltpu.async_copy(table_hbm.at[idx_a], rows_a, sem_a)]
    for j in range(1, NCHUNK + 1):
        if j < NCHUNK:
            idx_v, rows, sem = bufs[j % 2]
            pltpu.sync_copy(idx_hbm.at[pl.ds(base + j * CHUNK, CHUNK)], idx_v)
            copies.append(pltpu.async_copy(table_hbm.at[idx_v], rows, sem))
        _, prev_rows, _ = bufs[(j - 1) % 2]
        copies[j - 1].wait()
        pltpu.sync_copy(prev_rows,
                        out_hbm.at[pl.ds(base + (j - 1) * CHUNK, CHUNK)])


def kernel(x, W_down, b_down, W_in, b_in, codebook, W_out, b_out, W_up, b_up):
    xf = x.reshape(BN, IN_FEAT)
    idx, table = _indices_and_table(xf, W_down, b_down, W_in, b_in, codebook,
                                    W_out, b_out, W_up, b_up)
    out = _sc_gather(table, idx.reshape(BN), )
    out = out.reshape(B, N, OUT_FEAT)
    indices = idx.reshape(B, N)
    commit_loss = jnp.zeros((), dtype=jnp.float32)
    return (out, indices, commit_loss)


# speedup vs baseline: 1.6981x; 1.6981x over previous
import functools

import jax
import jax.numpy as jnp
from jax import lax
from jax.experimental import pallas as pl
from jax.experimental.pallas import tpu as pltpu
from jax.experimental.pallas import tpu_sc as plsc

B = 64
N = 1024
IN_FEAT = 192
HIDDEN = 128
CODE_DIM = 32
CODEBOOK_SIZE = 256
OUT_FEAT = 192

TB = 2048
BN = B * N
GRID = BN // TB

NC = 2
NS = 16
NW = NC * NS
TOK_PER_W = BN // NW
CHUNK = 256
NCHUNK = TOK_PER_W // CHUNK


def _tc_body(x_ref, wd_ref, bd_ref, wi_ref, bi_ref, cb_ref, wo_ref, bo_ref,
             wu_ref, bu_ref, idx_ref, table_ref):
    def mm(a, b, dims):
        return jax.lax.dot_general(a.astype(jnp.bfloat16),
                                   b.astype(jnp.bfloat16), dims,
                                   preferred_element_type=jnp.float32)

    @pl.when(pl.program_id(0) == 0)
    def _():
        cb0 = cb_ref[...]
        t0 = mm(cb0, wo_ref[...], (((1,), (1,)), ((), ()))) + bo_ref[...]
        t1 = mm(t0, wu_ref[...], (((1,), (1,)), ((), ()))) + bu_ref[...]
        table_ref[...] = jnp.clip(t1, -1.0, 1.0)

    x = x_ref[...]
    h = mm(x, wd_ref[...], (((1,), (1,)), ((), ()))) + bd_ref[...]
    z = mm(h, wi_ref[...], (((1,), (1,)), ((), ()))) + bi_ref[...]
    cb = cb_ref[...]
    scores = mm(z, cb, (((1,), (1,)), ((), ())))
    zz = jnp.sum(z * z, axis=1, keepdims=True)
    c2 = jax.lax.dot_general(jnp.ones((1, CODE_DIM), jnp.float32), cb * cb,
                             (((1,), (1,)), ((), ())),
                             precision=jax.lax.Precision.HIGHEST,
                             preferred_element_type=jnp.float32)
    dist = (zz - 2.0 * scores) + c2
    dmin = jnp.min(dist, axis=1, keepdims=True)
    lane = jax.lax.broadcasted_iota(jnp.int32, dist.shape, 1)
    idx_ref[...] = jnp.min(jnp.where(dist == dmin, lane, CODEBOOK_SIZE),
                           axis=1, keepdims=True)


def _indices_and_table(xf, W_down, b_down, W_in, b_in, codebook, W_out, b_out,
                       W_up, b_up):
    full = lambda shape: pl.BlockSpec(shape, lambda i: (0,) * len(shape))
    return pl.pallas_call(
        _tc_body,
        grid=(GRID,),
        in_specs=[
            pl.BlockSpec((TB, IN_FEAT), lambda i: (i, 0)),
            full((HIDDEN, IN_FEAT)),
            full((1, HIDDEN)),
            full((CODE_DIM, HIDDEN)),
            full((1, CODE_DIM)),
            full((CODEBOOK_SIZE, CODE_DIM)),
            full((HIDDEN, CODE_DIM)),
            full((1, HIDDEN)),
            full((OUT_FEAT, HIDDEN)),
            full((1, OUT_FEAT)),
        ],
        out_specs=[
            pl.BlockSpec((TB, 1), lambda i: (i, 0)),
            full((CODEBOOK_SIZE, OUT_FEAT)),
        ],
        out_shape=[
            jax.ShapeDtypeStruct((BN, 1), jnp.int32),
            jax.ShapeDtypeStruct((CODEBOOK_SIZE, OUT_FEAT), jnp.float32),
        ],
    )(xf, W_down, b_down.reshape(1, HIDDEN), W_in, b_in.reshape(1, CODE_DIM),
      codebook, W_out, b_out.reshape(1, HIDDEN), W_up,
      b_up.reshape(1, OUT_FEAT))


@functools.partial(
    pl.kernel,
    out_type=jax.ShapeDtypeStruct((BN, OUT_FEAT), jnp.float32),
    mesh=plsc.VectorSubcoreMesh(core_axis_name="c", subcore_axis_name="s",
                                num_cores=NC, num_subcores=NS),
    compiler_params=pltpu.CompilerParams(use_tc_tiling_on_sc=False),
    scratch_types=[
        pltpu.VMEM_SHARED((CODEBOOK_SIZE, OUT_FEAT), jnp.float32),
        pltpu.VMEM((CHUNK,), jnp.int32),
        pltpu.VMEM((CHUNK,), jnp.int32),
        pltpu.VMEM((CHUNK, OUT_FEAT), jnp.float32),
        pltpu.VMEM((CHUNK, OUT_FEAT), jnp.float32),
        pltpu.SemaphoreType.DMA,
        pltpu.SemaphoreType.DMA,
    ],
)
def _sc_gather(table_hbm, idx_hbm, out_hbm, table_sp, idx_a, idx_b, rows_a,
               rows_b, sem_a, sem_b):
    sid = lax.axis_index("s")
    wid = sid * NC + lax.axis_index("c")
    base = wid * TOK_PER_W

    @pl.when(sid == 0)
    def _():
        pltpu.sync_copy(table_hbm, table_sp)
    plsc.subcore_barrier()

    bufs = ((idx_a, rows_a, sem_a), (idx_b, rows_b, sem_b))
    pltpu.sync_copy(idx_hbm.at[pl.ds(base, CHUNK)], idx_a)
    copies = [pltpu.async_copy(table_sp.at[idx_a], rows_a, sem_a)]
    for j in range(1, NCHUNK + 1):
        if j < NCHUNK:
            idx_v, rows, sem = bufs[j % 2]
            pltpu.sync_copy(idx_hbm.at[pl.ds(base + j * CHUNK, CHUNK)], idx_v)
            copies.append(pltpu.async_copy(table_sp.at[idx_v], rows, sem))
        _, prev_rows, _ = bufs[(j - 1) % 2]
        copies[j - 1].wait()
        pltpu.sync_copy(prev_rows,
                        out_hbm.at[pl.ds(base + (j - 1) * CHUNK, CHUNK)])


def kernel(x, W_down, b_down, W_in, b_in, codebook, W_out, b_out, W_up, b_up):
    xf = x.reshape(BN, IN_FEAT)
    idx, table = _indices_and_table(xf, W_down, b_down, W_in, b_in, codebook,
                                    W_out, b_out, W_up, b_up)
    out = _sc_gather(table, idx.reshape(BN), )
    out = out.reshape(B, N, OUT_FEAT)
    indices = idx.reshape(B, N)
    commit_loss = jnp.zeros((), dtype=jnp.float32)
    return (out, indices, commit_loss)


# speedup vs baseline: 4.1028x; 2.4161x over previous
import jax
import jax.numpy as jnp
from jax.experimental import pallas as pl
from jax.experimental.pallas import tpu as pltpu

B = 64
N = 1024
IN_FEAT = 192
HIDDEN = 128
CODE_DIM = 32
CODEBOOK_SIZE = 256
OUT_FEAT = 192


def _body(xt_ref, wd_ref, bd_ref, wi_ref, bi_ref, cb_ref, wo_ref, bo_ref,
          wu_ref, bu_ref, out_ref, idx_ref, table_ref):
    def mm(a, b, dims):
        return jax.lax.dot_general(a.astype(jnp.bfloat16),
                                   b.astype(jnp.bfloat16), dims,
                                   preferred_element_type=jnp.float32)

    @pl.when(pl.program_id(0) == 0)
    def _():
        cb0 = cb_ref[...]
        t0 = mm(cb0, wo_ref[...], (((1,), (1,)), ((), ()))) + bo_ref[...]
        t1 = mm(t0, wu_ref[...], (((1,), (1,)), ((), ()))) + bu_ref[...]
        table_ref[...] = jnp.clip(t1, -1.0, 1.0)

    xt = xt_ref[0]
    h = mm(wd_ref[...], xt, (((1,), (0,)), ((), ()))) + bd_ref[...]
    z = mm(wi_ref[...], h, (((1,), (0,)), ((), ()))) + bi_ref[...]
    cb = cb_ref[...]
    scores = mm(cb, z, (((1,), (0,)), ((), ())))
    zz = jnp.sum(z * z, axis=0, keepdims=True)
    c2 = jax.lax.dot_general(cb * cb, jnp.ones((CODE_DIM, 1), jnp.float32),
                             (((1,), (0,)), ((), ())),
                             precision=jax.lax.Precision.HIGHEST,
                             preferred_element_type=jnp.float32)
    dist = (zz - 2.0 * scores) + c2
    dmin = jnp.min(dist, axis=0, keepdims=True)
    row = jax.lax.broadcasted_iota(jnp.int32, dist.shape, 0)
    idx = jnp.min(jnp.where(dist == dmin, row, CODEBOOK_SIZE), axis=0,
                  keepdims=True)
    idx_ref[0] = idx
    onehot = (row == idx).astype(jnp.float32)
    out_ref[0] = jax.lax.dot_general(table_ref[...], onehot,
                                     (((0,), (0,)), ((), ())),
                                     preferred_element_type=jnp.float32)


def kernel(x, W_down, b_down, W_in, b_in, codebook, W_out, b_out, W_up, b_up):
    xt = jnp.transpose(x, (0, 2, 1))
    full = lambda shape: pl.BlockSpec(shape, lambda i: (0,) * len(shape))
    out_t, idx = pl.pallas_call(
        _body,
        grid=(B,),
        in_specs=[
            pl.BlockSpec((1, IN_FEAT, N), lambda i: (i, 0, 0)),
            full((HIDDEN, IN_FEAT)),
            full((HIDDEN, 1)),
            full((CODE_DIM, HIDDEN)),
            full((CODE_DIM, 1)),
            full((CODEBOOK_SIZE, CODE_DIM)),
            full((HIDDEN, CODE_DIM)),
            full((1, HIDDEN)),
            full((OUT_FEAT, HIDDEN)),
            full((1, OUT_FEAT)),
        ],
        out_specs=[
            pl.BlockSpec((1, OUT_FEAT, N), lambda i: (i, 0, 0)),
            pl.BlockSpec((1, 1, N), lambda i: (i, 0, 0)),
        ],
        out_shape=[
            jax.ShapeDtypeStruct((B, OUT_FEAT, N), jnp.float32),
            jax.ShapeDtypeStruct((B, 1, N), jnp.int32),
        ],
        scratch_shapes=[pltpu.VMEM((CODEBOOK_SIZE, OUT_FEAT), jnp.float32)],
    )(xt, W_down, b_down.reshape(HIDDEN, 1), W_in, b_in.reshape(CODE_DIM, 1),
      codebook, W_out, b_out.reshape(1, HIDDEN), W_up, b_up.reshape(1, OUT_FEAT))
    out = jnp.transpose(out_t, (0, 2, 1))
    indices = idx.reshape(B, N)
    commit_loss = jnp.zeros((), dtype=jnp.float32)
    return (out, indices, commit_loss)


# speedup vs baseline: 6.0325x; 1.4703x over previous
import jax
import jax.numpy as jnp
from jax.experimental import pallas as pl
from jax.experimental.pallas import tpu as pltpu

B = 64
N = 1024
IN_FEAT = 192
HIDDEN = 128
CODE_DIM = 32
CODEBOOK_SIZE = 256
OUT_FEAT = 192
UNROLL = 2


def _body(xt_ref, wd_ref, bd_ref, wi_ref, bi_ref, cb_ref, wo_ref, bo_ref,
          wu_ref, bu_ref, out_ref, idx_ref, table_ref):
    def mm(a, b, dims):
        return jax.lax.dot_general(a.astype(jnp.bfloat16),
                                   b.astype(jnp.bfloat16), dims,
                                   preferred_element_type=jnp.float32)

    @pl.when(pl.program_id(0) == 0)
    def _():
        cb0 = cb_ref[...]
        t0 = mm(cb0, wo_ref[...], (((1,), (1,)), ((), ()))) + bo_ref[...]
        t1 = mm(t0, wu_ref[...], (((1,), (1,)), ((), ()))) + bu_ref[...]
        table_ref[...] = jnp.clip(t1, -1.0, 1.0)

    cb = cb_ref[...]
    c2 = jax.lax.dot_general(cb * cb, jnp.ones((CODE_DIM, 1), jnp.float32),
                             (((1,), (0,)), ((), ())),
                             precision=jax.lax.Precision.HIGHEST,
                             preferred_element_type=jnp.float32)
    table16 = table_ref[...].astype(jnp.bfloat16)
    for u in range(UNROLL):
        xt = xt_ref[u]
        h = mm(wd_ref[...], xt, (((1,), (0,)), ((), ()))) + bd_ref[...]
        z = mm(wi_ref[...], h, (((1,), (0,)), ((), ()))) + bi_ref[...]
        scores = mm(cb, z, (((1,), (0,)), ((), ())))
        zz = jnp.sum(z * z, axis=0, keepdims=True)
        dist = (zz - 2.0 * scores) + c2
        dmin = jnp.min(dist, axis=0, keepdims=True)
        row = jax.lax.broadcasted_iota(jnp.int32, dist.shape, 0)
        idx = jnp.min(jnp.where(dist == dmin, row, CODEBOOK_SIZE), axis=0,
                      keepdims=True)
        idx_ref[u] = idx
        onehot = (row == idx).astype(jnp.bfloat16)
        out_ref[u] = jax.lax.dot_general(table16, onehot,
                                         (((0,), (0,)), ((), ())),
                                         preferred_element_type=jnp.float32)


def kernel(x, W_down, b_down, W_in, b_in, codebook, W_out, b_out, W_up, b_up):
    xt = jnp.transpose(x, (0, 2, 1))
    full = lambda shape: pl.BlockSpec(shape, lambda i: (0,) * len(shape))
    out_t, idx = pl.pallas_call(
        _body,
        grid=(B // UNROLL,),
        in_specs=[
            pl.BlockSpec((UNROLL, IN_FEAT, N), lambda i: (i, 0, 0)),
            full((HIDDEN, IN_FEAT)),
            full((HIDDEN, 1)),
            full((CODE_DIM, HIDDEN)),
            full((CODE_DIM, 1)),
            full((CODEBOOK_SIZE, CODE_DIM)),
            full((HIDDEN, CODE_DIM)),
            full((1, HIDDEN)),
            full((OUT_FEAT, HIDDEN)),
            full((1, OUT_FEAT)),
        ],
        out_specs=[
            pl.BlockSpec((UNROLL, OUT_FEAT, N), lambda i: (i, 0, 0)),
            pl.BlockSpec((UNROLL, 1, N), lambda i: (i, 0, 0)),
        ],
        out_shape=[
            jax.ShapeDtypeStruct((B, OUT_FEAT, N), jnp.float32),
            jax.ShapeDtypeStruct((B, 1, N), jnp.int32),
        ],
        scratch_shapes=[pltpu.VMEM((CODEBOOK_SIZE, OUT_FEAT), jnp.float32)],
    )(xt, W_down, b_down.reshape(HIDDEN, 1), W_in, b_in.reshape(CODE_DIM, 1),
      codebook, W_out, b_out.reshape(1, HIDDEN), W_up, b_up.reshape(1, OUT_FEAT))
    out = jnp.transpose(out_t, (0, 2, 1))
    indices = idx.reshape(B, N)
    commit_loss = jnp.zeros((), dtype=jnp.float32)
    return (out, indices, commit_loss)


# speedup vs baseline: 6.4295x; 1.0658x over previous
import jax
import jax.numpy as jnp
from jax.experimental import pallas as pl
from jax.experimental.pallas import tpu as pltpu

B = 64
N = 1024
IN_FEAT = 192
HIDDEN = 128
CODE_DIM = 32
CODEBOOK_SIZE = 256
OUT_FEAT = 192
UNROLL = 4


def _body(xt_ref, wd_ref, bd_ref, wi_ref, bi_ref, cb_ref, wo_ref, bo_ref,
          wu_ref, bu_ref, out_ref, idx_ref, table_ref):
    def mm(a, b, dims):
        return jax.lax.dot_general(a.astype(jnp.bfloat16),
                                   b.astype(jnp.bfloat16), dims,
                                   preferred_element_type=jnp.float32)

    @pl.when(pl.program_id(0) == 0)
    def _():
        cb0 = cb_ref[...]
        t0 = mm(cb0, wo_ref[...], (((1,), (1,)), ((), ()))) + bo_ref[...]
        t1 = mm(t0, wu_ref[...], (((1,), (1,)), ((), ()))) + bu_ref[...]
        table_ref[...] = jnp.clip(t1, -1.0, 1.0)

    cb = cb_ref[...]
    c2 = jax.lax.dot_general(cb * cb, jnp.ones((CODE_DIM, 1), jnp.float32),
                             (((1,), (0,)), ((), ())),
                             precision=jax.lax.Precision.HIGHEST,
                             preferred_element_type=jnp.float32)
    table16 = table_ref[...].astype(jnp.bfloat16)
    for u in range(UNROLL):
        xt = xt_ref[u]
        h = mm(wd_ref[...], xt, (((1,), (0,)), ((), ()))) + bd_ref[...]
        z = mm(wi_ref[...], h, (((1,), (0,)), ((), ()))) + bi_ref[...]
        scores = mm(cb, z, (((1,), (0,)), ((), ())))
        zz = jnp.sum(z * z, axis=0, keepdims=True)
        dist = (zz - 2.0 * scores) + c2
        dmin = jnp.min(dist, axis=0, keepdims=True)
        row = jax.lax.broadcasted_iota(jnp.int32, dist.shape, 0)
        idx = jnp.min(jnp.where(dist == dmin, row, CODEBOOK_SIZE), axis=0,
                      keepdims=True)
        idx_ref[u] = idx
        onehot = (row == idx).astype(jnp.bfloat16)
        out_ref[u] = jax.lax.dot_general(table16, onehot,
                                         (((0,), (0,)), ((), ())),
                                         preferred_element_type=jnp.float32)


def kernel(x, W_down, b_down, W_in, b_in, codebook, W_out, b_out, W_up, b_up):
    xt = jnp.transpose(x, (0, 2, 1))
    full = lambda shape: pl.BlockSpec(shape, lambda i: (0,) * len(shape))
    out_t, idx = pl.pallas_call(
        _body,
        grid=(B // UNROLL,),
        in_specs=[
            pl.BlockSpec((UNROLL, IN_FEAT, N), lambda i: (i, 0, 0)),
            full((HIDDEN, IN_FEAT)),
            full((HIDDEN, 1)),
            full((CODE_DIM, HIDDEN)),
            full((CODE_DIM, 1)),
            full((CODEBOOK_SIZE, CODE_DIM)),
            full((HIDDEN, CODE_DIM)),
            full((1, HIDDEN)),
            full((OUT_FEAT, HIDDEN)),
            full((1, OUT_FEAT)),
        ],
        out_specs=[
            pl.BlockSpec((UNROLL, OUT_FEAT, N), lambda i: (i, 0, 0)),
            pl.BlockSpec((UNROLL, 1, N), lambda i: (i, 0, 0)),
        ],
        out_shape=[
            jax.ShapeDtypeStruct((B, OUT_FEAT, N), jnp.float32),
            jax.ShapeDtypeStruct((B, 1, N), jnp.int32),
        ],
        scratch_shapes=[pltpu.VMEM((CODEBOOK_SIZE, OUT_FEAT), jnp.float32)],
    )(xt, W_down, b_down.reshape(HIDDEN, 1), W_in, b_in.reshape(CODE_DIM, 1),
      codebook, W_out, b_out.reshape(1, HIDDEN), W_up, b_up.reshape(1, OUT_FEAT))
    out = jnp.transpose(out_t, (0, 2, 1))
    indices = idx.reshape(B, N)
    commit_loss = jnp.zeros((), dtype=jnp.float32)
    return (out, indices, commit_loss)
